# baseline (speedup 1.0000x reference)
import jax
import jax.numpy as jnp
from jax.experimental import pallas as pl
from jax.experimental.pallas import tpu as pltpu

B = 512
P = 196
H = 768
M = 512
NOPS = 8
ROWS_PER_OP = 4
ROWS_PER_STEP = NOPS * ROWS_PER_OP
NSTEP = B // ROWS_PER_STEP
INV_P = 1.0 / P


def _mean_fifo_body(ts_ref, *refs):
    in_refs = refs[:NOPS]
    mem_ref = refs[NOPS]
    ts_out_ref = refs[NOPS + 1]
    parts = [jnp.sum(r[:], axis=1) * INV_P for r in in_refs]
    mem_ref[:] = jnp.concatenate(parts, axis=0)
    ts_out_ref[:] = ts_ref[:]


def _in_spec(k):
    return pl.BlockSpec((ROWS_PER_OP, P, H), lambda i, k=k: (i * NOPS + k, 0, 0))


def kernel(states, timestamp, memory_states, memory_timestamps):
    ts3 = timestamp.astype(jnp.int32).reshape(NSTEP, 1, ROWS_PER_STEP)
    new_mem, new_ts = pl.pallas_call(
        _mean_fifo_body,
        grid=(NSTEP,),
        in_specs=[pl.BlockSpec((1, 1, ROWS_PER_STEP), lambda i: (i, 0, 0))]
        + [_in_spec(k) for k in range(NOPS)],
        out_specs=[
            pl.BlockSpec((ROWS_PER_STEP, H), lambda i: (i, 0)),
            pl.BlockSpec((1, 1, ROWS_PER_STEP), lambda i: (i, 0, 0)),
        ],
        out_shape=[
            jax.ShapeDtypeStruct((M, H), jnp.float32),
            jax.ShapeDtypeStruct((NSTEP, 1, ROWS_PER_STEP), jnp.int32),
        ],
        compiler_params=pltpu.CompilerParams(
            dimension_semantics=("parallel",),
        ),
    )(ts3, *([states] * NOPS))
    new_ts = new_ts.reshape(B).astype(memory_timestamps.dtype)
    new_valid = jnp.ones((M,), dtype=jnp.bool_)
    new_ptr = jnp.full((1,), B % M, dtype=jnp.int32)
    new_count = jnp.full((1,), min(B, M), dtype=jnp.int32)
    return (new_mem, new_ts, new_valid, new_ptr, new_count)
